# baseline (speedup 1.0000x reference)
import functools

import jax
import jax.numpy as jnp
from jax import lax
from jax.experimental import pallas as pl
from jax.experimental.pallas import tpu as pltpu
from jax.experimental.pallas import tpu_sc as plsc

_B = 16384
_VOCAB_F1 = 3
_VOCAB_F2 = 11
_EMB_DIM = 10
_H1 = 20
_NCOMBO = _VOCAB_F1 * _VOCAB_F2
_TBL = 48

_NC, _NS, _L = 2, 16, 16
_NW = _NC * _NS
_BPW = _B // _NW

_OFF_EMB = 0
_OFF_W1 = _OFF_EMB + _VOCAB_F2 * _EMB_DIM
_OFF_B1 = _OFF_W1 + 13 * _H1
_OFF_W2 = _OFF_B1 + _H1
_OFF_B2 = _OFF_W2 + _H1
_WLEN = 416


@functools.partial(
    pl.kernel,
    out_type=jax.ShapeDtypeStruct((_B,), jnp.float32),
    mesh=plsc.VectorSubcoreMesh(core_axis_name="c", subcore_axis_name="s"),
    compiler_params=pltpu.CompilerParams(needs_layout_passes=False),
    scratch_types=[
        pltpu.VMEM((_BPW,), jnp.int32),
        pltpu.VMEM((_BPW,), jnp.int32),
        pltpu.VMEM((_WLEN,), jnp.float32),
        pltpu.VMEM((_H1 * _L,), jnp.float32),
        pltpu.VMEM((_TBL,), jnp.float32),
        pltpu.VMEM((_BPW,), jnp.float32),
        pltpu.SemaphoreType.DMA,
    ],
)
def _sc_kernel(f1_hbm, f2_hbm, wts_hbm, out_hbm,
               f1_v, f2_v, wts_v, m_v, tbl_v, out_v, sem):
    def splat(off):
        return plsc.load_gather(wts_v, [jnp.full((_L,), off, jnp.int32)])

    wid = lax.axis_index("s") * _NC + lax.axis_index("c")
    base = wid * _BPW
    cw = pltpu.async_copy(wts_hbm, wts_v, sem)
    c1 = pltpu.async_copy(f1_hbm.at[pl.ds(base, _BPW)], f1_v, sem)
    c2 = pltpu.async_copy(f2_hbm.at[pl.ds(base, _BPW)], f2_v, sem)
    cw.wait()

    lanes = lax.iota(jnp.int32, _L)

    bln = jnp.minimum(lanes, _VOCAB_F2 - 1)
    emb_cols = [plsc.load_gather(wts_v, [bln * _EMB_DIM + k])
                for k in range(_EMB_DIM)]
    for j in range(_H1):
        acc = jnp.zeros((_L,), jnp.float32)
        for k in range(_EMB_DIM):
            acc = acc + emb_cols[k] * splat(_OFF_W1 + (_VOCAB_F1 + k) * _H1 + j)
        m_v[pl.ds(j * _L, _L)] = acc

    avec, bvec, cacc = [], [], []
    for t in range(3):
        c = lanes + t * _L
        q = c // _VOCAB_F2
        avec.append(jnp.minimum(q, _VOCAB_F1 - 1))
        bvec.append(c - q * _VOCAB_F2)
        cacc.append(jnp.zeros((_L,), jnp.float32))
    for j in range(_H1):
        b1j = splat(_OFF_B1 + j)
        w2j = splat(_OFF_W2 + j)
        for t in range(3):
            w1aj = plsc.load_gather(wts_v, [avec[t] * _H1 + (_OFF_W1 + j)])
            mbj = plsc.load_gather(m_v, [bvec[t] + j * _L])
            h = jnp.maximum(w1aj + mbj + b1j, 0.0)
            cacc[t] = cacc[t] + h * w2j
    b2s = splat(_OFF_B2)
    for t in range(3):
        y = cacc[t] + b2s
        tbl_v[pl.ds(t * _L, _L)] = y / (1.0 + jnp.abs(y))

    c1.wait()
    c2.wait()
    half = _BPW // 2
    for i in range(half // _L):
        s = pl.ds(i * _L, _L)
        idx = f1_v[s] * _VOCAB_F2 + f2_v[s]
        out_v[s] = plsc.load_gather(tbl_v, [idx])
    o1 = pltpu.async_copy(out_v.at[pl.ds(0, half)],
                          out_hbm.at[pl.ds(base, half)], sem)
    for i in range(half // _L, _BPW // _L):
        s = pl.ds(i * _L, _L)
        idx = f1_v[s] * _VOCAB_F2 + f2_v[s]
        out_v[s] = plsc.load_gather(tbl_v, [idx])
    o2 = pltpu.async_copy(out_v.at[pl.ds(half, half)],
                          out_hbm.at[pl.ds(base + half, half)], sem)
    o1.wait()
    o2.wait()


def kernel(f1, f2, emb_f2, W1, b1, W2, b2):
    f1 = f1.astype(jnp.int32)
    f2 = f2.astype(jnp.int32)
    wts = jnp.concatenate([
        emb_f2.reshape(-1), W1.reshape(-1), b1, W2.reshape(-1), b2,
        jnp.zeros((_WLEN - (_OFF_B2 + 1),), jnp.float32),
    ])
    out = _sc_kernel(f1, f2, wts)
    return out.reshape(_B, 1)


# speedup vs baseline: 1.0309x; 1.0309x over previous
import functools

import jax
import jax.numpy as jnp
from jax import lax
from jax.experimental import pallas as pl
from jax.experimental.pallas import tpu as pltpu
from jax.experimental.pallas import tpu_sc as plsc

_B = 16384
_VOCAB_F1 = 3
_VOCAB_F2 = 11
_EMB_DIM = 10
_H1 = 20
_NCOMBO = _VOCAB_F1 * _VOCAB_F2
_TBL = 64

_NC, _NS, _L = 2, 16, 16
_NW = _NC * _NS
_BPW = _B // _NW


def _table_body(emb_ref, w1_ref, b1_ref, w2_ref, b2_ref, out_ref):
    c = lax.broadcasted_iota(jnp.int32, (_TBL, 1), 0)
    a = c // _VOCAB_F2
    b = c % _VOCAB_F2
    oh1 = (a == lax.broadcasted_iota(jnp.int32, (_TBL, _VOCAB_F1), 1)).astype(jnp.float32)
    oh2 = (b == lax.broadcasted_iota(jnp.int32, (_TBL, _VOCAB_F2), 1)).astype(jnp.float32)
    emb = jnp.dot(oh2, emb_ref[...], preferred_element_type=jnp.float32)
    h = (jnp.dot(oh1, w1_ref[: _VOCAB_F1, :], preferred_element_type=jnp.float32)
         + jnp.dot(emb, w1_ref[_VOCAB_F1:, :], preferred_element_type=jnp.float32)
         + b1_ref[...])
    h = jnp.maximum(h, 0.0)
    y = jnp.dot(h, w2_ref[...], preferred_element_type=jnp.float32) + b2_ref[...]
    out_ref[...] = y / (1.0 + jnp.abs(y))


_table_call = pl.pallas_call(
    _table_body,
    out_shape=jax.ShapeDtypeStruct((_TBL, 1), jnp.float32),
)


@functools.partial(
    pl.kernel,
    out_type=jax.ShapeDtypeStruct((_B,), jnp.float32),
    mesh=plsc.VectorSubcoreMesh(core_axis_name="c", subcore_axis_name="s"),
    compiler_params=pltpu.CompilerParams(needs_layout_passes=False),
    scratch_types=[
        pltpu.VMEM((_BPW,), jnp.int32),
        pltpu.VMEM((_BPW,), jnp.int32),
        pltpu.VMEM((_TBL,), jnp.float32),
        pltpu.VMEM((_BPW,), jnp.float32),
        pltpu.SemaphoreType.DMA,
    ],
)
def _sc_gather(f1_hbm, f2_hbm, tbl_hbm, out_hbm, f1_v, f2_v, tbl_v, out_v, sem):
    wid = lax.axis_index("s") * _NC + lax.axis_index("c")
    base = wid * _BPW
    c1 = pltpu.async_copy(tbl_hbm, tbl_v, sem)
    c2 = pltpu.async_copy(f1_hbm.at[pl.ds(base, _BPW)], f1_v, sem)
    c3 = pltpu.async_copy(f2_hbm.at[pl.ds(base, _BPW)], f2_v, sem)
    c1.wait()
    c2.wait()
    c3.wait()
    half = _BPW // 2
    for i in range(half // _L):
        s = pl.ds(i * _L, _L)
        idx = f1_v[s] * _VOCAB_F2 + f2_v[s]
        out_v[s] = plsc.load_gather(tbl_v, [idx])
    o1 = pltpu.async_copy(out_v.at[pl.ds(0, half)],
                          out_hbm.at[pl.ds(base, half)], sem)
    for i in range(half // _L, _BPW // _L):
        s = pl.ds(i * _L, _L)
        idx = f1_v[s] * _VOCAB_F2 + f2_v[s]
        out_v[s] = plsc.load_gather(tbl_v, [idx])
    o2 = pltpu.async_copy(out_v.at[pl.ds(half, half)],
                          out_hbm.at[pl.ds(base + half, half)], sem)
    o1.wait()
    o2.wait()


def kernel(f1, f2, emb_f2, W1, b1, W2, b2):
    f1 = f1.astype(jnp.int32)
    f2 = f2.astype(jnp.int32)
    tbl = _table_call(emb_f2, W1, b1.reshape(1, _H1), W2, b2.reshape(1, 1))
    out = _sc_gather(f1, f2, tbl.reshape(_TBL))
    return out.reshape(_B, 1)


# speedup vs baseline: 1.1041x; 1.0710x over previous
import functools

import jax
import jax.numpy as jnp
from jax import lax
from jax.experimental import pallas as pl
from jax.experimental.pallas import tpu as pltpu
from jax.experimental.pallas import tpu_sc as plsc

_B = 16384
_VOCAB_F1 = 3
_VOCAB_F2 = 11
_EMB_DIM = 10
_H1 = 20
_NCOMBO = _VOCAB_F1 * _VOCAB_F2
_TBL = 64

_NC, _NS, _L = 2, 16, 16
_NW = _NC * _NS
_BPW = _B // _NW


def _table_body(emb_ref, w1_ref, b1_ref, w2_ref, b2_ref, out_ref):
    c = lax.broadcasted_iota(jnp.int32, (_TBL, 1), 0)
    a = c // _VOCAB_F2
    b = c % _VOCAB_F2
    oh1 = (a == lax.broadcasted_iota(jnp.int32, (_TBL, _VOCAB_F1), 1)).astype(jnp.float32)
    oh2 = (b == lax.broadcasted_iota(jnp.int32, (_TBL, _VOCAB_F2), 1)).astype(jnp.float32)
    emb = jnp.dot(oh2, emb_ref[...], preferred_element_type=jnp.float32)
    h = (jnp.dot(oh1, w1_ref[: _VOCAB_F1, :], preferred_element_type=jnp.float32)
         + jnp.dot(emb, w1_ref[_VOCAB_F1:, :], preferred_element_type=jnp.float32)
         + jnp.broadcast_to(b1_ref[...], (_TBL, _H1)))
    h = jnp.maximum(h, 0.0)
    y = lax.dot_general(w2_ref[...], h, (((0,), (1,)), ((), ())),
                        preferred_element_type=jnp.float32)
    y = y + jnp.broadcast_to(b2_ref[...], (1, _TBL))
    out_ref[...] = y / (1.0 + jnp.abs(y))


_table_call = pl.pallas_call(
    _table_body,
    out_shape=jax.ShapeDtypeStruct((1, _TBL), jnp.float32),
)


@functools.partial(
    pl.kernel,
    out_type=jax.ShapeDtypeStruct((_B,), jnp.float32),
    mesh=plsc.VectorSubcoreMesh(core_axis_name="c", subcore_axis_name="s"),
    compiler_params=pltpu.CompilerParams(needs_layout_passes=False),
    scratch_types=[
        pltpu.VMEM((_BPW,), jnp.int32),
        pltpu.VMEM((_BPW,), jnp.int32),
        pltpu.VMEM((_TBL,), jnp.float32),
        pltpu.VMEM((_BPW,), jnp.float32),
        pltpu.SemaphoreType.DMA,
    ],
)
def _sc_gather(f1_hbm, f2_hbm, tbl_hbm, out_hbm, f1_v, f2_v, tbl_v, out_v, sem):
    wid = lax.axis_index("s") * _NC + lax.axis_index("c")
    base = wid * _BPW
    c1 = pltpu.async_copy(tbl_hbm, tbl_v, sem)
    c2 = pltpu.async_copy(f1_hbm.at[pl.ds(base, _BPW)], f1_v, sem)
    c3 = pltpu.async_copy(f2_hbm.at[pl.ds(base, _BPW)], f2_v, sem)
    c1.wait()
    c2.wait()
    c3.wait()
    half = _BPW // 2
    for i in range(half // _L):
        s = pl.ds(i * _L, _L)
        idx = f1_v[s] * _VOCAB_F2 + f2_v[s]
        out_v[s] = plsc.load_gather(tbl_v, [idx])
    o1 = pltpu.async_copy(out_v.at[pl.ds(0, half)],
                          out_hbm.at[pl.ds(base, half)], sem)
    for i in range(half // _L, _BPW // _L):
        s = pl.ds(i * _L, _L)
        idx = f1_v[s] * _VOCAB_F2 + f2_v[s]
        out_v[s] = plsc.load_gather(tbl_v, [idx])
    o2 = pltpu.async_copy(out_v.at[pl.ds(half, half)],
                          out_hbm.at[pl.ds(base + half, half)], sem)
    o1.wait()
    o2.wait()


def kernel(f1, f2, emb_f2, W1, b1, W2, b2):
    f1 = f1.astype(jnp.int32)
    f2 = f2.astype(jnp.int32)
    tbl = _table_call(emb_f2, W1, b1, W2, b2)
    out = _sc_gather(f1, f2, tbl.reshape(_TBL))
    return out.reshape(_B, 1)


# speedup vs baseline: 1.1047x; 1.0005x over previous
import functools

import jax
import jax.numpy as jnp
from jax import lax
from jax.experimental import pallas as pl
from jax.experimental.pallas import tpu as pltpu
from jax.experimental.pallas import tpu_sc as plsc

_B = 16384
_VOCAB_F1 = 3
_VOCAB_F2 = 11
_EMB_DIM = 10
_H1 = 20
_NCOMBO = _VOCAB_F1 * _VOCAB_F2
_TBL = 64

_NC, _NS, _L = 2, 16, 16
_NW = _NC * _NS
_BPW = _B // _NW


def _table_body(emb_ref, w1_ref, b1_ref, w2_ref, b2_ref, out_ref):
    c = lax.broadcasted_iota(jnp.int32, (_TBL, 1), 0)
    a = c // _VOCAB_F2
    b = c % _VOCAB_F2
    oh1 = (a == lax.broadcasted_iota(jnp.int32, (_TBL, _VOCAB_F1), 1)).astype(jnp.float32)
    oh2 = (b == lax.broadcasted_iota(jnp.int32, (_TBL, _VOCAB_F2), 1)).astype(jnp.float32)
    emb = jnp.dot(oh2, emb_ref[...], preferred_element_type=jnp.float32)
    h = (jnp.dot(oh1, w1_ref[: _VOCAB_F1, :], preferred_element_type=jnp.float32)
         + jnp.dot(emb, w1_ref[_VOCAB_F1:, :], preferred_element_type=jnp.float32)
         + jnp.broadcast_to(b1_ref[...], (_TBL, _H1)))
    h = jnp.maximum(h, 0.0)
    y = lax.dot_general(w2_ref[...], h, (((0,), (1,)), ((), ())),
                        preferred_element_type=jnp.float32)
    y = y + jnp.broadcast_to(b2_ref[...], (1, _TBL))
    out_ref[...] = y / (1.0 + jnp.abs(y))


_table_call = pl.pallas_call(
    _table_body,
    out_shape=jax.ShapeDtypeStruct((1, _TBL), jnp.float32),
)


@functools.partial(
    pl.kernel,
    out_type=jax.ShapeDtypeStruct((_B,), jnp.float32),
    mesh=plsc.VectorSubcoreMesh(core_axis_name="c", subcore_axis_name="s"),
    compiler_params=pltpu.CompilerParams(needs_layout_passes=False),
    scratch_types=[
        pltpu.VMEM((_BPW,), jnp.int32),
        pltpu.VMEM((_BPW,), jnp.int32),
        pltpu.VMEM((_TBL,), jnp.float32),
        pltpu.VMEM((_BPW,), jnp.float32),
        pltpu.SemaphoreType.DMA,
    ],
)
def _sc_gather(f1_hbm, f2_hbm, tbl_hbm, out_hbm, f1_v, f2_v, tbl_v, out_v, sem):
    wid = lax.axis_index("s") * _NC + lax.axis_index("c")
    base = wid * _BPW
    half = _BPW // 2
    c1 = pltpu.async_copy(tbl_hbm, tbl_v, sem)
    c2a = pltpu.async_copy(f1_hbm.at[pl.ds(base, half)], f1_v.at[pl.ds(0, half)], sem)
    c3a = pltpu.async_copy(f2_hbm.at[pl.ds(base, half)], f2_v.at[pl.ds(0, half)], sem)
    c2b = pltpu.async_copy(f1_hbm.at[pl.ds(base + half, half)],
                           f1_v.at[pl.ds(half, half)], sem)
    c3b = pltpu.async_copy(f2_hbm.at[pl.ds(base + half, half)],
                           f2_v.at[pl.ds(half, half)], sem)
    c1.wait()
    c2a.wait()
    c3a.wait()
    for i in range(half // _L):
        s = pl.ds(i * _L, _L)
        idx = f1_v[s] * _VOCAB_F2 + f2_v[s]
        out_v[s] = plsc.load_gather(tbl_v, [idx])
    o1 = pltpu.async_copy(out_v.at[pl.ds(0, half)],
                          out_hbm.at[pl.ds(base, half)], sem)
    c2b.wait()
    c3b.wait()
    for i in range(half // _L, _BPW // _L):
        s = pl.ds(i * _L, _L)
        idx = f1_v[s] * _VOCAB_F2 + f2_v[s]
        out_v[s] = plsc.load_gather(tbl_v, [idx])
    o2 = pltpu.async_copy(out_v.at[pl.ds(half, half)],
                          out_hbm.at[pl.ds(base + half, half)], sem)
    o1.wait()
    o2.wait()


def kernel(f1, f2, emb_f2, W1, b1, W2, b2):
    f1 = f1.astype(jnp.int32)
    f2 = f2.astype(jnp.int32)
    tbl = _table_call(emb_f2, W1, b1, W2, b2)
    out = _sc_gather(f1, f2, tbl.reshape(_TBL))
    return out.reshape(_B, 1)
